# baseline (speedup 1.0000x reference)
import functools

import jax
import jax.numpy as jnp
from jax import lax
from jax.experimental import pallas as pl
from jax.experimental.pallas import tpu as pltpu
from jax.experimental.pallas import tpu_sc as plsc

B = 16384
NNUM = 13
NCAT = 26
VOCAB = 100000
EDIM = 16
L1 = 200
L2 = 50
NCLS = 2

_NC = 2
_NS = 16
_NW = _NC * _NS
_ROWS = B * NCAT
_RPW = _ROWS // _NW
_CHUNK = 3328
_NCHUNK = _RPW // _CHUNK


def _sc_gather(flat_table, gidx):
    mesh = plsc.VectorSubcoreMesh(
        core_axis_name="c", subcore_axis_name="s",
        num_cores=_NC, num_subcores=_NS)

    @functools.partial(
        pl.kernel,
        out_type=jax.ShapeDtypeStruct((_ROWS, EDIM), jnp.float32),
        mesh=mesh,
        scratch_types=[
            pltpu.VMEM((_RPW,), jnp.int32),
            pltpu.VMEM((_CHUNK, EDIM), jnp.float32),
            pltpu.SemaphoreType.DMA,
        ],
        compiler_params=pltpu.CompilerParams(use_tc_tiling_on_sc=False),
    )
    def gather_kernel(flat_hbm, idx_hbm, out_hbm, idx_v, rows_v, sem):
        wid = lax.axis_index("s") * _NC + lax.axis_index("c")
        base = wid * _RPW
        pltpu.sync_copy(idx_hbm.at[pl.ds(base, _RPW)], idx_v)

        def body(i, carry):
            off = i * _CHUNK
            pltpu.async_copy(
                flat_hbm.at[idx_v.at[pl.ds(off, _CHUNK)]], rows_v, sem
            ).wait()
            pltpu.sync_copy(rows_v, out_hbm.at[pl.ds(base + off, _CHUNK)])
            return carry

        lax.fori_loop(0, _NCHUNK, body, 0)

    return gather_kernel(flat_table, gidx)


_VPF = VOCAB + 96
_GPF = _VPF // 8
_NTC = _VPF // 128
_TASKS = NCAT * _NTC
_TPW = -(-_TASKS // _NW)


def _sc_format(tables3, tailpad):
    mesh = plsc.VectorSubcoreMesh(
        core_axis_name="c", subcore_axis_name="s",
        num_cores=_NC, num_subcores=_NS)

    @functools.partial(
        pl.kernel,
        out_type=jax.ShapeDtypeStruct((NCAT * _GPF, 128), jnp.float32),
        mesh=mesh,
        scratch_types=[
            pltpu.VMEM((EDIM, 128), jnp.float32),
            pltpu.VMEM((EDIM, 128), jnp.float32),
        ],
        compiler_params=pltpu.CompilerParams(use_tc_tiling_on_sc=True,
                                             needs_layout_passes=False),
    )
    def fmt_kernel(tab_hbm, tail_hbm, out_hbm, inb, outb):
        wid = lax.axis_index("s") * _NC + lax.axis_index("c")
        lanes = lax.iota(jnp.int32, 16)

        def task(i, carry):
            t = wid + _NW * i
            f = t // _NTC
            tc = t % _NTC
            is_tail = tc == _NTC - 1
            g0 = f * _GPF + 16 * tc

            @pl.when((t < _TASKS) & is_tail)
            def _():
                pltpu.sync_copy(tail_hbm.at[f], outb)
                pltpu.sync_copy(outb, out_hbm.at[pl.ds(g0, 16)])

            @pl.when((t < _TASKS) & jnp.logical_not(is_tail))
            def _():
                pltpu.sync_copy(tab_hbm.at[f, :, pl.ds(128 * tc, 128)], inb)

                def col(v, c):
                    row = plsc.load_gather(inb, [lanes, jnp.full((16,), v,
                                                                 jnp.int32)])
                    plsc.store_scatter(
                        outb, [jnp.full((16,), v // 8, jnp.int32),
                               16 * (v % 8) + lanes], row)
                    return c
                lax.fori_loop(0, 128, col, 0)
                pltpu.sync_copy(outb, out_hbm.at[pl.ds(g0, 16)])
            return carry

        lax.fori_loop(0, _TPW, task, 0)

    return fmt_kernel(tables3, tailpad)


_BLK = 2048


def _mlp_body(nums_ref, emb_ref, w1n_ref, w1e_ref, b1_ref, w2_ref, b2_ref,
              wp_ref, bp_ref, out_ref):
    x = jnp.dot(nums_ref[...], w1n_ref[...], preferred_element_type=jnp.float32)
    x = x + jnp.dot(emb_ref[...], w1e_ref[...],
                    preferred_element_type=jnp.float32)
    h = jnp.maximum(x + b1_ref[...], 0.0)
    h = jnp.maximum(
        jnp.dot(h, w2_ref[...], preferred_element_type=jnp.float32)
        + b2_ref[...], 0.0)
    logits = (jnp.dot(h, wp_ref[...], preferred_element_type=jnp.float32)
              + bp_ref[...])
    m = jnp.max(logits, axis=-1, keepdims=True)
    e = jnp.exp(logits - m)
    out_ref[...] = e / jnp.sum(e, axis=-1, keepdims=True)


def _tc_mlp(nums, emb, w1n, w1e, b1, w2, b2, wp, bp, interpret=False):
    fixed = lambda i: (0, 0)
    return pl.pallas_call(
        _mlp_body,
        grid=(B // _BLK,),
        in_specs=[
            pl.BlockSpec((_BLK, NNUM), lambda i: (i, 0)),
            pl.BlockSpec((_BLK, NCAT * EDIM), lambda i: (i, 0)),
            pl.BlockSpec((NNUM, L1), fixed),
            pl.BlockSpec((NCAT * EDIM, L1), fixed),
            pl.BlockSpec((1, L1), fixed),
            pl.BlockSpec((L1, L2), fixed),
            pl.BlockSpec((1, L2), fixed),
            pl.BlockSpec((L2, NCLS), fixed),
            pl.BlockSpec((1, NCLS), fixed),
        ],
        out_specs=pl.BlockSpec((_BLK, NCLS), lambda i: (i, 0)),
        out_shape=jax.ShapeDtypeStruct((B, NCLS), jnp.float32),
        interpret=interpret,
    )(nums, emb, w1n, w1e, b1, w2, b2, wp, bp)


def kernel(nums, cats, tables, W1, b1, W2, b2, Wp, bp):
    grouped3 = jnp.transpose(tables, (0, 2, 1))
    tailpad = jnp.pad(tables[:, VOCAB - 32:, :].reshape(NCAT, 4, 128),
                      ((0, 0), (0, 12), (0, 0)))
    flat_table = _sc_format(grouped3, tailpad).reshape(NCAT * _VPF, EDIM)
    offsets = (jnp.arange(NCAT, dtype=jnp.int32) * _VPF)[None, :]
    gidx = (cats + offsets).reshape(-1)
    emb = _sc_gather(flat_table, gidx).reshape(B, NCAT * EDIM)
    return _tc_mlp(nums, emb, W1[:NNUM], W1[NNUM:], b1.reshape(1, L1),
                   W2, b2.reshape(1, L2), Wp, bp.reshape(1, NCLS))


# speedup vs baseline: 1.2711x; 1.2711x over previous
import functools

import jax
import jax.numpy as jnp
from jax import lax
from jax.experimental import pallas as pl
from jax.experimental.pallas import tpu as pltpu
from jax.experimental.pallas import tpu_sc as plsc

B = 16384
NNUM = 13
NCAT = 26
VOCAB = 100000
EDIM = 16
L1 = 200
L2 = 50
NCLS = 2

_NC = 2
_NS = 16
_NW = _NC * _NS
_ROWS = B * NCAT
_RPW = _ROWS // _NW
_CHUNK = 3328
_NCHUNK = _RPW // _CHUNK


def _sc_gather(flat_table, gidx):
    mesh = plsc.VectorSubcoreMesh(
        core_axis_name="c", subcore_axis_name="s",
        num_cores=_NC, num_subcores=_NS)

    @functools.partial(
        pl.kernel,
        out_type=jax.ShapeDtypeStruct((_ROWS, EDIM), jnp.float32),
        mesh=mesh,
        scratch_types=[
            pltpu.VMEM((_RPW,), jnp.int32),
            pltpu.VMEM((_CHUNK, EDIM), jnp.float32),
            pltpu.SemaphoreType.DMA,
        ],
        compiler_params=pltpu.CompilerParams(use_tc_tiling_on_sc=False),
    )
    def gather_kernel(flat_hbm, idx_hbm, out_hbm, idx_v, rows_v, sem):
        wid = lax.axis_index("s") * _NC + lax.axis_index("c")
        base = wid * _RPW
        pltpu.sync_copy(idx_hbm.at[pl.ds(base, _RPW)], idx_v)

        def body(i, carry):
            off = i * _CHUNK
            pltpu.async_copy(
                flat_hbm.at[idx_v.at[pl.ds(off, _CHUNK)]], rows_v, sem
            ).wait()
            pltpu.sync_copy(rows_v, out_hbm.at[pl.ds(base + off, _CHUNK)])
            return carry

        lax.fori_loop(0, _NCHUNK, body, 0)

    return gather_kernel(flat_table, gidx)


_VPF = VOCAB + 96
_GPF = _VPF // 8
_NTC = _VPF // 128
_TASKS = NCAT * _NTC
_TPW = -(-_TASKS // _NW)


def _sc_format(tables3, tailpad):
    mesh = plsc.VectorSubcoreMesh(
        core_axis_name="c", subcore_axis_name="s",
        num_cores=_NC, num_subcores=_NS)

    @functools.partial(
        pl.kernel,
        out_type=jax.ShapeDtypeStruct((NCAT * _GPF, 128), jnp.float32),
        mesh=mesh,
        scratch_types=[
            pltpu.VMEM((EDIM, 128), jnp.float32),
            pltpu.VMEM((EDIM, 128), jnp.float32),
            pltpu.VMEM((EDIM, 128), jnp.float32),
            pltpu.VMEM((EDIM, 128), jnp.float32),
            pltpu.SemaphoreType.DMA,
            pltpu.SemaphoreType.DMA,
            pltpu.SemaphoreType.DMA,
            pltpu.SemaphoreType.DMA,
        ],
        compiler_params=pltpu.CompilerParams(use_tc_tiling_on_sc=True,
                                             needs_layout_passes=False),
    )
    def fmt_kernel(tab_hbm, tail_hbm, out_hbm,
                   inb0, inb1, outb0, outb1, sin0, sin1, sout0, sout1):
        wid = lax.axis_index("s") * _NC + lax.axis_index("c")
        lanes = lax.iota(jnp.int32, 16)
        inbs, outbs = (inb0, inb1), (outb0, outb1)
        sins, souts = (sin0, sin1), (sout0, sout1)

        def valid(j):
            return (j < _TPW) & (wid + _NW * j < _TASKS)

        def start_in(j, b):
            t = wid + _NW * j
            f = t // _NTC
            tc = t % _NTC

            @pl.when(valid(j) & (tc == _NTC - 1))
            def _():
                pltpu.async_copy(tail_hbm.at[f], inbs[b], sins[b])

            @pl.when(valid(j) & (tc < _NTC - 1))
            def _():
                pltpu.async_copy(tab_hbm.at[f, :, pl.ds(128 * tc, 128)],
                                 inbs[b], sins[b])

        start_in(0, 0)

        def body(i2, carry):
            for b in range(2):
                j = 2 * i2 + b
                start_in(j + 1, 1 - b)

                @pl.when(valid(j))
                def _(b=b, j=j):
                    pltpu.make_async_copy(
                        tab_hbm.at[0, :, pl.ds(0, 128)], inbs[b],
                        sins[b]).wait()

                    @pl.when(j >= 2)
                    def _():
                        pltpu.make_async_copy(
                            tab_hbm.at[0, :, pl.ds(0, 128)], outbs[b],
                            souts[b]).wait()

                    for v in range(128):
                        row = plsc.load_gather(
                            inbs[b], [lanes, jnp.full((16,), v, jnp.int32)])
                        plsc.store_scatter(
                            outbs[b], [jnp.full((16,), v // 8, jnp.int32),
                                       jnp.full((16,), 16 * (v % 8),
                                                jnp.int32) + lanes], row)

                    t = wid + _NW * j
                    g0 = (t // _NTC) * _GPF + 16 * (t % _NTC)
                    pltpu.async_copy(outbs[b], out_hbm.at[pl.ds(g0, 16)],
                                     souts[b])
            return carry

        lax.fori_loop(0, _TPW // 2, body, 0)
        for b in range(2):
            @pl.when(valid(_TPW - 2 + b))
            def _(b=b):
                pltpu.make_async_copy(
                    tab_hbm.at[0, :, pl.ds(0, 128)], outbs[b],
                    souts[b]).wait()

    return fmt_kernel(tables3, tailpad)


_BLK = 2048


def _mlp_body(nums_ref, emb_ref, w1n_ref, w1e_ref, b1_ref, w2_ref, b2_ref,
              wp_ref, bp_ref, out_ref):
    x = jnp.dot(nums_ref[...], w1n_ref[...], preferred_element_type=jnp.float32)
    x = x + jnp.dot(emb_ref[...], w1e_ref[...],
                    preferred_element_type=jnp.float32)
    h = jnp.maximum(x + b1_ref[...], 0.0)
    h = jnp.maximum(
        jnp.dot(h, w2_ref[...], preferred_element_type=jnp.float32)
        + b2_ref[...], 0.0)
    logits = (jnp.dot(h, wp_ref[...], preferred_element_type=jnp.float32)
              + bp_ref[...])
    m = jnp.max(logits, axis=-1, keepdims=True)
    e = jnp.exp(logits - m)
    out_ref[...] = e / jnp.sum(e, axis=-1, keepdims=True)


def _tc_mlp(nums, emb, w1n, w1e, b1, w2, b2, wp, bp, interpret=False):
    fixed = lambda i: (0, 0)
    return pl.pallas_call(
        _mlp_body,
        grid=(B // _BLK,),
        in_specs=[
            pl.BlockSpec((_BLK, NNUM), lambda i: (i, 0)),
            pl.BlockSpec((_BLK, NCAT * EDIM), lambda i: (i, 0)),
            pl.BlockSpec((NNUM, L1), fixed),
            pl.BlockSpec((NCAT * EDIM, L1), fixed),
            pl.BlockSpec((1, L1), fixed),
            pl.BlockSpec((L1, L2), fixed),
            pl.BlockSpec((1, L2), fixed),
            pl.BlockSpec((L2, NCLS), fixed),
            pl.BlockSpec((1, NCLS), fixed),
        ],
        out_specs=pl.BlockSpec((_BLK, NCLS), lambda i: (i, 0)),
        out_shape=jax.ShapeDtypeStruct((B, NCLS), jnp.float32),
        interpret=interpret,
    )(nums, emb, w1n, w1e, b1, w2, b2, wp, bp)


def kernel(nums, cats, tables, W1, b1, W2, b2, Wp, bp):
    grouped3 = jnp.transpose(tables, (0, 2, 1))
    tailpad = jnp.pad(jnp.transpose(tables[:, VOCAB - 32:, :], (0, 2, 1)),
                      ((0, 0), (0, 0), (0, 96)))
    flat_table = _sc_format(grouped3, tailpad).reshape(NCAT * _VPF, EDIM)
    offsets = (jnp.arange(NCAT, dtype=jnp.int32) * _VPF)[None, :]
    gidx = (cats + offsets).reshape(-1)
    emb = _sc_gather(flat_table, gidx).reshape(B, NCAT * EDIM)
    return _tc_mlp(nums, emb, W1[:NNUM], W1[NNUM:], b1.reshape(1, L1),
                   W2, b2.reshape(1, L2), Wp, bp.reshape(1, NCLS))


# speedup vs baseline: 1.4536x; 1.1436x over previous
import functools

import jax
import jax.numpy as jnp
from jax import lax
from jax.experimental import pallas as pl
from jax.experimental.pallas import tpu as pltpu
from jax.experimental.pallas import tpu_sc as plsc

B = 16384
NNUM = 13
NCAT = 26
VOCAB = 100000
EDIM = 16
L1 = 200
L2 = 50
NCLS = 2

_NC = 2
_NS = 16
_NW = _NC * _NS
_ROWS = B * NCAT
_RPW = _ROWS // _NW
_CHUNK = 3328
_NCHUNK = _RPW // _CHUNK


def _sc_gather(flat_table, gidx):
    mesh = plsc.VectorSubcoreMesh(
        core_axis_name="c", subcore_axis_name="s",
        num_cores=_NC, num_subcores=_NS)

    @functools.partial(
        pl.kernel,
        out_type=jax.ShapeDtypeStruct((_ROWS, EDIM), jnp.float32),
        mesh=mesh,
        scratch_types=[
            pltpu.VMEM((_RPW,), jnp.int32),
            pltpu.VMEM((_CHUNK, EDIM), jnp.float32),
            pltpu.SemaphoreType.DMA,
        ],
        compiler_params=pltpu.CompilerParams(use_tc_tiling_on_sc=False),
    )
    def gather_kernel(flat_hbm, idx_hbm, out_hbm, idx_v, rows_v, sem):
        wid = lax.axis_index("s") * _NC + lax.axis_index("c")
        base = wid * _RPW
        pltpu.sync_copy(idx_hbm.at[pl.ds(base, _RPW)], idx_v)

        def body(i, carry):
            off = i * _CHUNK
            pltpu.async_copy(
                flat_hbm.at[idx_v.at[pl.ds(off, _CHUNK)]], rows_v, sem
            ).wait()
            pltpu.sync_copy(rows_v, out_hbm.at[pl.ds(base + off, _CHUNK)])
            return carry

        lax.fori_loop(0, _NCHUNK, body, 0)

    return gather_kernel(flat_table, gidx)


_VPF = VOCAB + 96
_GPF = _VPF // 8
_WW = 512
_NU = VOCAB // _WW
_KPF = _NU + 2
_TASKS = NCAT * _KPF
_TPW = -(-_TASKS // _NW)
_NB = 4
_TPAD = -(-_TPW // _NB) * _NB


def _sc_format(tables3, tailpad):
    mesh = plsc.VectorSubcoreMesh(
        core_axis_name="c", subcore_axis_name="s",
        num_cores=_NC, num_subcores=_NS)

    @functools.partial(
        pl.kernel,
        out_type=jax.ShapeDtypeStruct((NCAT * _GPF, 128), jnp.float32),
        mesh=mesh,
        scratch_types=(
            [pltpu.VMEM((EDIM, _WW), jnp.float32)] * _NB
            + [pltpu.VMEM((_WW // 8, 128), jnp.float32)] * _NB
            + [pltpu.SemaphoreType.DMA] * (2 * _NB)
        ),
        compiler_params=pltpu.CompilerParams(use_tc_tiling_on_sc=True,
                                             needs_layout_passes=False),
    )
    def fmt_kernel(tab_hbm, tail_hbm, out_hbm, *bufs):
        inbs = bufs[:_NB]
        outbs = bufs[_NB:2 * _NB]
        sins = bufs[2 * _NB:3 * _NB]
        souts = bufs[3 * _NB:]
        wid = lax.axis_index("s") * _NC + lax.axis_index("c")
        lanes = lax.iota(jnp.int32, 16)

        def valid(j):
            return (j < _TPW) & (wid + _NW * j < _TASKS)

        def fu(j):
            t = wid + _NW * j
            return t // _KPF, t % _KPF

        def start_in(j, b):
            f, u = fu(j)

            @pl.when(valid(j) & (u < _NU))
            def _():
                pltpu.async_copy(tab_hbm.at[f, :, pl.ds(_WW * u, _WW)],
                                 inbs[b], sins[b])

            @pl.when(valid(j) & (u == _NU))
            def _():
                pltpu.async_copy(tab_hbm.at[f, :, pl.ds(_NU * _WW, 128)],
                                 inbs[b].at[:, pl.ds(0, 128)], sins[b])

            @pl.when(valid(j) & (u == _NU + 1))
            def _():
                pltpu.async_copy(tail_hbm.at[f],
                                 inbs[b].at[:, pl.ds(0, 128)], sins[b])

        def wait_in(j, b):
            _, u = fu(j)

            @pl.when(u < _NU)
            def _():
                pltpu.make_async_copy(tab_hbm.at[0, :, pl.ds(0, _WW)],
                                      inbs[b], sins[b]).wait()

            @pl.when(u >= _NU)
            def _():
                pltpu.make_async_copy(tab_hbm.at[0, :, pl.ds(0, 128)],
                                      inbs[b].at[:, pl.ds(0, 128)],
                                      sins[b]).wait()

        def start_out(j, b):
            f, u = fu(j)
            g0 = f * _GPF + jnp.where(u < _NU, (_WW // 8) * u,
                                      jnp.where(u == _NU, _GPF - 32,
                                                _GPF - 16))

            @pl.when(u < _NU)
            def _():
                pltpu.async_copy(outbs[b], out_hbm.at[pl.ds(g0, _WW // 8)],
                                 souts[b])

            @pl.when(u >= _NU)
            def _():
                pltpu.async_copy(outbs[b].at[pl.ds(0, 16)],
                                 out_hbm.at[pl.ds(g0, 16)], souts[b])

        def wait_out(j, b):
            @pl.when(valid(j))
            def _():
                _, u = fu(j)

                @pl.when(u < _NU)
                def _():
                    pltpu.make_async_copy(
                        tab_hbm.at[0, :, pl.ds(0, _WW)], outbs[b],
                        souts[b]).wait()

                @pl.when(u >= _NU)
                def _():
                    pltpu.make_async_copy(
                        tab_hbm.at[0, :, pl.ds(0, 128)],
                        outbs[b].at[pl.ds(0, 16)], souts[b]).wait()

        for b in range(_NB - 1):
            start_in(b, b)

        def body(i2, carry):
            for b in range(_NB):
                j = _NB * i2 + b
                start_in(j + _NB - 1, (b + _NB - 1) % _NB)

                @pl.when(valid(j))
                def _(b=b, j=j):
                    wait_in(j, b)
                    @pl.when(j >= _NB)
                    def _():
                        wait_out(j - _NB, b)

                    def col8(q, c):
                        base = q * 8
                        rows = jnp.full((16,), q, jnp.int32)
                        for r in range(8):
                            row = plsc.load_gather(
                                inbs[b],
                                [lanes, jnp.full((16,), base, jnp.int32) + r])
                            plsc.store_scatter(
                                outbs[b],
                                [rows, jnp.full((16,), 16 * r, jnp.int32)
                                 + lanes], row)
                        return c
                    lax.fori_loop(0, _WW // 8, col8, 0)
                    start_out(j, b)
            return carry

        lax.fori_loop(0, _TPAD // _NB, body, 0)
        for b in range(_NB):
            wait_out(_TPAD - _NB + b, b)

    return fmt_kernel(tables3, tailpad)


_BLK = 2048


def _mlp_body(nums_ref, emb_ref, w1n_ref, w1e_ref, b1_ref, w2_ref, b2_ref,
              wp_ref, bp_ref, out_ref):
    x = jnp.dot(nums_ref[...], w1n_ref[...], preferred_element_type=jnp.float32)
    x = x + jnp.dot(emb_ref[...], w1e_ref[...],
                    preferred_element_type=jnp.float32)
    h = jnp.maximum(x + b1_ref[...], 0.0)
    h = jnp.maximum(
        jnp.dot(h, w2_ref[...], preferred_element_type=jnp.float32)
        + b2_ref[...], 0.0)
    logits = (jnp.dot(h, wp_ref[...], preferred_element_type=jnp.float32)
              + bp_ref[...])
    m = jnp.max(logits, axis=-1, keepdims=True)
    e = jnp.exp(logits - m)
    out_ref[...] = e / jnp.sum(e, axis=-1, keepdims=True)


def _tc_mlp(nums, emb, w1n, w1e, b1, w2, b2, wp, bp, interpret=False):
    fixed = lambda i: (0, 0)
    return pl.pallas_call(
        _mlp_body,
        grid=(B // _BLK,),
        in_specs=[
            pl.BlockSpec((_BLK, NNUM), lambda i: (i, 0)),
            pl.BlockSpec((_BLK, NCAT * EDIM), lambda i: (i, 0)),
            pl.BlockSpec((NNUM, L1), fixed),
            pl.BlockSpec((NCAT * EDIM, L1), fixed),
            pl.BlockSpec((1, L1), fixed),
            pl.BlockSpec((L1, L2), fixed),
            pl.BlockSpec((1, L2), fixed),
            pl.BlockSpec((L2, NCLS), fixed),
            pl.BlockSpec((1, NCLS), fixed),
        ],
        out_specs=pl.BlockSpec((_BLK, NCLS), lambda i: (i, 0)),
        out_shape=jax.ShapeDtypeStruct((B, NCLS), jnp.float32),
        interpret=interpret,
    )(nums, emb, w1n, w1e, b1, w2, b2, wp, bp)


def kernel(nums, cats, tables, W1, b1, W2, b2, Wp, bp):
    grouped3 = jnp.transpose(tables, (0, 2, 1))
    tailpad = jnp.pad(jnp.transpose(tables[:, VOCAB - 32:, :], (0, 2, 1)),
                      ((0, 0), (0, 0), (0, 96)))
    flat_table = _sc_format(grouped3, tailpad).reshape(NCAT * _VPF, EDIM)
    offsets = (jnp.arange(NCAT, dtype=jnp.int32) * _VPF)[None, :]
    gidx = (cats + offsets).reshape(-1)
    emb = _sc_gather(flat_table, gidx).reshape(B, NCAT * EDIM)
    return _tc_mlp(nums, emb, W1[:NNUM], W1[NNUM:], b1.reshape(1, L1),
                   W2, b2.reshape(1, L2), Wp, bp.reshape(1, NCLS))


# speedup vs baseline: 6.7687x; 4.6565x over previous
import functools

import jax
import jax.numpy as jnp
from jax import lax
from jax.experimental import pallas as pl
from jax.experimental.pallas import tpu as pltpu
from jax.experimental.pallas import tpu_sc as plsc

B = 16384
NNUM = 13
NCAT = 26
VOCAB = 100000
EDIM = 16
L1 = 200
L2 = 50
NCLS = 2

_NC = 2
_NS = 16
_NW = _NC * _NS
_ROWS = B * NCAT
_RPW = _ROWS // _NW
_CHUNK = 3328
_NCHUNK = _RPW // _CHUNK


def _sc_gather(flat_table, gidx):
    mesh = plsc.VectorSubcoreMesh(
        core_axis_name="c", subcore_axis_name="s",
        num_cores=_NC, num_subcores=_NS)

    @functools.partial(
        pl.kernel,
        out_type=jax.ShapeDtypeStruct((_ROWS, EDIM), jnp.float32),
        mesh=mesh,
        scratch_types=[
            pltpu.VMEM((_RPW,), jnp.int32),
            pltpu.VMEM((_CHUNK, EDIM), jnp.float32),
            pltpu.SemaphoreType.DMA,
        ],
        compiler_params=pltpu.CompilerParams(use_tc_tiling_on_sc=False),
    )
    def gather_kernel(flat_hbm, idx_hbm, out_hbm, idx_v, rows_v, sem):
        wid = lax.axis_index("s") * _NC + lax.axis_index("c")
        base = wid * _RPW
        pltpu.sync_copy(idx_hbm.at[pl.ds(base, _RPW)], idx_v)

        def body(i, carry):
            off = i * _CHUNK
            pltpu.async_copy(
                flat_hbm.at[idx_v.at[pl.ds(off, _CHUNK)]], rows_v, sem
            ).wait()
            pltpu.sync_copy(rows_v, out_hbm.at[pl.ds(base + off, _CHUNK)])
            return carry

        lax.fori_loop(0, _NCHUNK, body, 0)

    return gather_kernel(flat_table, gidx)


_VPF = VOCAB + 96
_GPF = _VPF // 8
_WW = 512
_NU = VOCAB // _WW
_KPF = _NU + 2
_TASKS = NCAT * _KPF
_TPW = -(-_TASKS // _NW)
_NB = 4
_TPAD = -(-_TPW // _NB) * _NB


def _sc_format(tables3, tailpad):
    mesh = plsc.VectorSubcoreMesh(
        core_axis_name="c", subcore_axis_name="s",
        num_cores=_NC, num_subcores=_NS)

    @functools.partial(
        pl.kernel,
        out_type=jax.ShapeDtypeStruct((NCAT * _GPF, 128), jnp.float32),
        mesh=mesh,
        scratch_types=(
            [pltpu.VMEM((EDIM, _WW), jnp.float32)] * _NB
            + [pltpu.VMEM((_WW // 8, 128), jnp.float32)] * _NB
            + [pltpu.SemaphoreType.DMA] * (2 * _NB)
        ),
        compiler_params=pltpu.CompilerParams(use_tc_tiling_on_sc=True,
                                             needs_layout_passes=False),
    )
    def fmt_kernel(tab_hbm, tail_hbm, out_hbm, *bufs):
        inbs = bufs[:_NB]
        outbs = bufs[_NB:2 * _NB]
        sins = bufs[2 * _NB:3 * _NB]
        souts = bufs[3 * _NB:]
        wid = lax.axis_index("s") * _NC + lax.axis_index("c")
        lanes = lax.iota(jnp.int32, 16)

        def valid(j):
            return (j < _TPW) & (wid + _NW * j < _TASKS)

        def fu(j):
            t = wid + _NW * j
            return t // _KPF, t % _KPF

        def start_in(j, b):
            f, u = fu(j)

            @pl.when(valid(j) & (u < _NU))
            def _():
                pltpu.async_copy(tab_hbm.at[f, :, pl.ds(_WW * u, _WW)],
                                 inbs[b], sins[b])

            @pl.when(valid(j) & (u == _NU))
            def _():
                pltpu.async_copy(tab_hbm.at[f, :, pl.ds(_NU * _WW, 128)],
                                 inbs[b].at[:, pl.ds(0, 128)], sins[b])

            @pl.when(valid(j) & (u == _NU + 1))
            def _():
                pltpu.async_copy(tail_hbm.at[f],
                                 inbs[b].at[:, pl.ds(0, 128)], sins[b])

        def wait_in(j, b):
            _, u = fu(j)

            @pl.when(u < _NU)
            def _():
                pltpu.make_async_copy(tab_hbm.at[0, :, pl.ds(0, _WW)],
                                      inbs[b], sins[b]).wait()

            @pl.when(u >= _NU)
            def _():
                pltpu.make_async_copy(tab_hbm.at[0, :, pl.ds(0, 128)],
                                      inbs[b].at[:, pl.ds(0, 128)],
                                      sins[b]).wait()

        def start_out(j, b):
            f, u = fu(j)
            g0 = f * _GPF + jnp.where(u < _NU, (_WW // 8) * u,
                                      jnp.where(u == _NU, _GPF - 32,
                                                _GPF - 16))

            @pl.when(u < _NU)
            def _():
                pltpu.async_copy(outbs[b], out_hbm.at[pl.ds(g0, _WW // 8)],
                                 souts[b])

            @pl.when(u >= _NU)
            def _():
                pltpu.async_copy(outbs[b].at[pl.ds(0, 16)],
                                 out_hbm.at[pl.ds(g0, 16)], souts[b])

        def wait_out(j, b):
            @pl.when(valid(j))
            def _():
                _, u = fu(j)

                @pl.when(u < _NU)
                def _():
                    pltpu.make_async_copy(
                        tab_hbm.at[0, :, pl.ds(0, _WW)], outbs[b],
                        souts[b]).wait()

                @pl.when(u >= _NU)
                def _():
                    pltpu.make_async_copy(
                        tab_hbm.at[0, :, pl.ds(0, 128)],
                        outbs[b].at[pl.ds(0, 16)], souts[b]).wait()

        for b in range(_NB - 1):
            start_in(b, b)

        def body(i2, carry):
            for b in range(_NB):
                j = _NB * i2 + b
                start_in(j + _NB - 1, (b + _NB - 1) % _NB)

                @pl.when(valid(j))
                def _(b=b, j=j):
                    wait_in(j, b)
                    @pl.when(j >= _NB)
                    def _():
                        wait_out(j - _NB, b)

                    @functools.partial(plsc.parallel_loop, 0, _WW // 8,
                                       unroll=4)
                    def col8(q):
                        base = q * 8
                        rows = jnp.full((16,), q, jnp.int32)
                        for r in range(8):
                            row = plsc.load_gather(
                                inbs[b],
                                [lanes, jnp.full((16,), base, jnp.int32) + r])
                            plsc.store_scatter(
                                outbs[b],
                                [rows, jnp.full((16,), 16 * r, jnp.int32)
                                 + lanes], row)
                    start_out(j, b)
            return carry

        lax.fori_loop(0, _TPAD // _NB, body, 0)
        for b in range(_NB):
            wait_out(_TPAD - _NB + b, b)

    return fmt_kernel(tables3, tailpad)


_BLK = 2048


def _mlp_body(nums_ref, emb_ref, w1n_ref, w1e_ref, b1_ref, w2_ref, b2_ref,
              wp_ref, bp_ref, out_ref):
    x = jnp.dot(nums_ref[...], w1n_ref[...], preferred_element_type=jnp.float32)
    x = x + jnp.dot(emb_ref[...], w1e_ref[...],
                    preferred_element_type=jnp.float32)
    h = jnp.maximum(x + b1_ref[...], 0.0)
    h = jnp.maximum(
        jnp.dot(h, w2_ref[...], preferred_element_type=jnp.float32)
        + b2_ref[...], 0.0)
    logits = (jnp.dot(h, wp_ref[...], preferred_element_type=jnp.float32)
              + bp_ref[...])
    m = jnp.max(logits, axis=-1, keepdims=True)
    e = jnp.exp(logits - m)
    out_ref[...] = e / jnp.sum(e, axis=-1, keepdims=True)


def _tc_mlp(nums, emb, w1n, w1e, b1, w2, b2, wp, bp, interpret=False):
    fixed = lambda i: (0, 0)
    return pl.pallas_call(
        _mlp_body,
        grid=(B // _BLK,),
        in_specs=[
            pl.BlockSpec((_BLK, NNUM), lambda i: (i, 0)),
            pl.BlockSpec((_BLK, NCAT * EDIM), lambda i: (i, 0)),
            pl.BlockSpec((NNUM, L1), fixed),
            pl.BlockSpec((NCAT * EDIM, L1), fixed),
            pl.BlockSpec((1, L1), fixed),
            pl.BlockSpec((L1, L2), fixed),
            pl.BlockSpec((1, L2), fixed),
            pl.BlockSpec((L2, NCLS), fixed),
            pl.BlockSpec((1, NCLS), fixed),
        ],
        out_specs=pl.BlockSpec((_BLK, NCLS), lambda i: (i, 0)),
        out_shape=jax.ShapeDtypeStruct((B, NCLS), jnp.float32),
        interpret=interpret,
    )(nums, emb, w1n, w1e, b1, w2, b2, wp, bp)


def kernel(nums, cats, tables, W1, b1, W2, b2, Wp, bp):
    grouped3 = jnp.transpose(tables, (0, 2, 1))
    tailpad = jnp.pad(jnp.transpose(tables[:, VOCAB - 32:, :], (0, 2, 1)),
                      ((0, 0), (0, 0), (0, 96)))
    flat_table = _sc_format(grouped3, tailpad).reshape(NCAT * _VPF, EDIM)
    offsets = (jnp.arange(NCAT, dtype=jnp.int32) * _VPF)[None, :]
    gidx = (cats + offsets).reshape(-1)
    emb = _sc_gather(flat_table, gidx).reshape(B, NCAT * EDIM)
    return _tc_mlp(nums, emb, W1[:NNUM], W1[NNUM:], b1.reshape(1, L1),
                   W2, b2.reshape(1, L2), Wp, bp.reshape(1, NCLS))
